# baseline (speedup 1.0000x reference)
import jax
import jax.numpy as jnp
from jax.experimental import pallas as pl
from jax.experimental.pallas import tpu as pltpu

NUM_HEADS = 16
NUM_BINS = 32
MAX_DIST = 1000000.0
T = 2048
BI = 256
JC = 128


def _body(pos_q_ref, pos_k_ref, packed_ref, out_ref, bins_ref):
    p = pl.program_id(1)

    @pl.when(p == 0)
    def _compute_bins():
        q = pos_q_ref[0, :]
        dmax = jnp.log1p(jnp.float32(MAX_DIST))
        for j0 in range(0, T, JC):
            k = pos_k_ref[0, j0:j0 + JC]
            d = jnp.abs(q[:, None] - k[None, :])
            d = jnp.clip(d, 0.0, MAX_DIST)
            d = jnp.log1p(d)
            bins_ref[:, j0:j0 + JC] = (d / dmax * (NUM_BINS - 1)).astype(
                jnp.int32)

    tab = jnp.broadcast_to(packed_ref[p, :][None, :], (8, NUM_BINS))
    for j0 in range(0, T, JC):
        for r in range(0, BI, 8):
            br = bins_ref[r:r + 8, j0:j0 + JC]
            g = jnp.take_along_axis(tab, br, axis=-1)
            gu = g.astype(jnp.uint32)
            lo = jax.lax.bitcast_convert_type(gu << 16, jnp.float32)
            hi = jax.lax.bitcast_convert_type(gu & jnp.uint32(0xFFFF0000),
                                              jnp.float32)
            out_ref[0, 0, r:r + 8, j0:j0 + JC] = lo
            out_ref[0, 1, r:r + 8, j0:j0 + JC] = hi


@jax.jit
def kernel(pos, bias):
    b16 = jax.lax.bitcast_convert_type(bias.astype(jnp.bfloat16),
                                       jnp.uint16).astype(jnp.uint32)
    packed = (b16[0::2, :] | (b16[1::2, :] << 16)).astype(jnp.int32)
    return pl.pallas_call(
        _body,
        grid=(T // BI, NUM_HEADS // 2),
        in_specs=[
            pl.BlockSpec((1, BI), lambda i, p: (0, i)),
            pl.BlockSpec((1, T), lambda i, p: (0, 0)),
            pl.BlockSpec((NUM_HEADS // 2, NUM_BINS), lambda i, p: (0, 0)),
        ],
        out_specs=pl.BlockSpec((1, 2, BI, T), lambda i, p: (0, p, i, 0)),
        out_shape=jax.ShapeDtypeStruct((1, NUM_HEADS, T, T), jnp.float32),
        scratch_shapes=[pltpu.VMEM((BI, T), jnp.int32)],
        compiler_params=pltpu.CompilerParams(
            dimension_semantics=("arbitrary", "arbitrary"),
        ),
    )(pos, pos, packed)


# speedup vs baseline: 1.3135x; 1.3135x over previous
import jax
import jax.numpy as jnp
from jax.experimental import pallas as pl
from jax.experimental.pallas import tpu as pltpu

NUM_HEADS = 16
NUM_BINS = 32
MAX_DIST = 1000000.0
T = 2048
BI = 128
JC = 128


def _body(pos_q_ref, pos_k_ref, bias_ref, out_ref):
    q = pos_q_ref[0, :]
    dmax = jnp.log1p(jnp.float32(MAX_DIST))
    tabs = [
        jnp.broadcast_to(bias_ref[h, :][None, :], (8, NUM_BINS))
        for h in range(NUM_HEADS)
    ]
    for j0 in range(0, T, JC):
        k = pos_k_ref[0, j0:j0 + JC]
        d = jnp.abs(q[:, None] - k[None, :])
        d = jnp.clip(d, 0.0, MAX_DIST)
        d = jnp.log1p(d)
        bins = (d / dmax * (NUM_BINS - 1)).astype(jnp.int32)
        for r in range(0, BI, 8):
            br = bins[r:r + 8, :]
            for h in range(NUM_HEADS):
                out_ref[0, h, r:r + 8, j0:j0 + JC] = jnp.take_along_axis(
                    tabs[h], br, axis=-1)


@jax.jit
def kernel(pos, bias):
    return pl.pallas_call(
        _body,
        grid=(T // BI,),
        in_specs=[
            pl.BlockSpec((1, BI), lambda i: (0, i)),
            pl.BlockSpec((1, T), lambda i: (0, 0)),
            pl.BlockSpec((NUM_HEADS, NUM_BINS), lambda i: (0, 0)),
        ],
        out_specs=pl.BlockSpec((1, NUM_HEADS, BI, T), lambda i: (0, 0, i, 0)),
        out_shape=jax.ShapeDtypeStruct((1, NUM_HEADS, T, T), jnp.float32),
        compiler_params=pltpu.CompilerParams(
            dimension_semantics=("parallel",),
        ),
    )(pos, pos, bias)
